# Initial kernel scaffold; baseline (speedup 1.0000x reference)
import jax
import jax.numpy as jnp
from jax.experimental import pallas as pl


def kernel(w1, b1, w2, b2, w3, b3, w4, b4, w5, b5, x):
    raise NotImplementedError("write your pallas kernel here")


# speedup vs baseline: 22.1354x; 22.1354x over previous
import numpy as np
import jax
import jax.numpy as jnp
from jax.experimental import pallas as pl
from jax.experimental.pallas import tpu as pltpu

PC1 = 1280
PC2 = 512
NB = 256


def _build_idx1():
    Z1 = 25 * 128
    idx = np.full((784, 4 * PC1), Z1, np.int32)
    ph = np.arange(14)
    for par in range(4):
        dh, dw = par // 2, par % 2
        for i in range(5):
            for j in range(5):
                u = 2 * ph + dh + i - 2
                v = 2 * ph + dw + j - 2
                mu, mv = (u >= 0) & (u < 28), (v >= 0) & (v < 28)
                pu, pv = ph[mu], ph[mv]
                rows = (u[mu][:, None] * 28 + v[mv][None, :]).ravel()
                colbase = par * PC1 + ((pu[:, None] * 14 + pv[None, :]) * 6).ravel()
                tap = i * 5 + j
                idx[rows[:, None], colbase[:, None] + np.arange(6)[None, :]] = (
                    tap * 128 + np.arange(6)[None, :])
    return idx


def _build_idx2():
    Z2 = 3200 * 128
    idx = np.full((PC1, 4 * PC2), Z2, np.int32)
    ph = np.arange(5)
    for par in range(4):
        dh, dw = par // 2, par % 2
        for i in range(5):
            for j in range(5):
                qr = 2 * ph + dh + i
                qc = 2 * ph + dw + j
                p1 = (qr[:, None] * 14 + qc[None, :]).ravel()
                cols = par * PC2 + ((ph[:, None] * 5 + ph[None, :]) * 16).ravel()
                tap = i * 5 + j
                for ci in range(6):
                    rows = p1 * 6 + ci
                    idx[np.repeat(rows, 16).reshape(25, 16),
                        cols[:, None] + np.arange(16)[None, :]] = (
                        (tap * 128 + ci) * 128 + np.arange(16)[None, :])
    return idx


_IDX1 = _build_idx1()
_IDX2 = _build_idx2()
_IDXB1 = np.where(np.arange(PC1) < 1176, np.arange(PC1) % 6, 6).astype(np.int32)
_IDXB2 = np.where(np.arange(PC2) < 400, np.arange(PC2) % 16, 16).astype(np.int32)


def _fused_body(x_ref, k1_ref, b1_ref, k2_ref, b2_ref,
                w3_ref, b3_ref, w4_ref, b4_ref, w5_ref, b5_ref, o_ref):
    x = x_ref[...]
    k1 = k1_ref[...]
    f32 = jnp.float32
    y = jnp.dot(x, k1[:, 0:PC1], preferred_element_type=f32)
    for p in range(1, 4):
        y = jnp.maximum(y, jnp.dot(x, k1[:, p * PC1:(p + 1) * PC1],
                                   preferred_element_type=f32))
    h1 = jnp.maximum(y + b1_ref[...], 0.0)

    k2 = k2_ref[...]
    y2 = jnp.dot(h1, k2[:, 0:PC2], preferred_element_type=f32)
    for p in range(1, 4):
        y2 = jnp.maximum(y2, jnp.dot(h1, k2[:, p * PC2:(p + 1) * PC2],
                                     preferred_element_type=f32))
    h2 = jnp.maximum(y2 + b2_ref[...], 0.0)

    h3 = jnp.maximum(jnp.dot(h2, w3_ref[...], preferred_element_type=f32)
                     + b3_ref[...], 0.0)
    h4 = jnp.maximum(jnp.dot(h3, w4_ref[...], preferred_element_type=f32)
                     + b4_ref[...], 0.0)
    o_ref[...] = (jnp.dot(h4, w5_ref[...], preferred_element_type=f32)
                  + b5_ref[...]).astype(o_ref.dtype)


def kernel(w1, b1, w2, b2, w3, b3, w4, b4, w5, b5, x):
    n = x.shape[0]
    xf = x.reshape(n, 28 * 28)

    w1f = jnp.concatenate([w1.reshape(-1), jnp.zeros((1,), jnp.float32)])
    k1 = w1f[_IDX1]
    w2f = jnp.concatenate([w2.reshape(-1), jnp.zeros((1,), jnp.float32)])
    k2 = w2f[_IDX2]
    b1t = b1[0][_IDXB1][None, :]
    b2t = b2[0][_IDXB2][None, :]
    w3p = jnp.pad(w3.reshape(25, 128, 128)[:, :16, :].reshape(400, 128),
                  ((0, PC2 - 400), (0, 0)))

    npad = ((n + NB - 1) // NB) * NB
    if npad != n:
        xf = jnp.pad(xf, ((0, npad - n), (0, 0)))
    g = npad // NB

    full = lambda i: (0, 0)
    cost = pl.CostEstimate(
        flops=2 * npad * (784 * 4 * PC1 + PC1 * 4 * PC2 + PC2 * 128 + 2 * 128 * 128),
        transcendentals=0,
        bytes_accessed=4 * (npad * 784 + npad * 128 + 784 * 4 * PC1
                            + PC1 * 4 * PC2 + PC2 * 128 + 2 * 128 * 128))

    out = pl.pallas_call(
        _fused_body,
        out_shape=jax.ShapeDtypeStruct((npad, 128), jnp.float32),
        grid=(g,),
        in_specs=[pl.BlockSpec((NB, 784), lambda i: (i, 0)),
                  pl.BlockSpec((784, 4 * PC1), full),
                  pl.BlockSpec((1, PC1), full),
                  pl.BlockSpec((PC1, 4 * PC2), full),
                  pl.BlockSpec((1, PC2), full),
                  pl.BlockSpec((PC2, 128), full),
                  pl.BlockSpec((1, 128), full),
                  pl.BlockSpec((128, 128), full),
                  pl.BlockSpec((1, 128), full),
                  pl.BlockSpec((128, 128), full),
                  pl.BlockSpec((1, 128), full)],
        out_specs=pl.BlockSpec((NB, 128), lambda i: (i, 0)),
        compiler_params=pltpu.CompilerParams(
            dimension_semantics=("parallel",),
            vmem_limit_bytes=100 * 1024 * 1024),
        cost_estimate=cost,
    )(xf, k1, b1t, k2, b2t, w3p, b3, w4, b4, w5, b5)
    return out[:n, :10]
